# Initial kernel scaffold; baseline (speedup 1.0000x reference)
import jax
import jax.numpy as jnp
from jax.experimental import pallas as pl


def kernel(features, edge_index, W_self0, W_neigh0, b0, W_self1, W_neigh1, b1):
    raise NotImplementedError("write your pallas kernel here")


# speedup vs baseline: 5.8946x; 5.8946x over previous
import functools

import jax
import jax.numpy as jnp
from jax import lax
from jax.experimental import pallas as pl
from jax.experimental.pallas import tpu as pltpu
from jax.experimental.pallas import tpu_sc as plsc

_N = 10000
_E = 320000
_D = 128
_H = 128
_C = 40

_NC = 2
_NS = 16
_K = 128
_NBLK = -(-_E // (_NC * _NS * _K))
_EPAD = _NC * _NS * _NBLK * _K
_NACC = 10240
_RPT = _NACC // _NS
_W0 = 144
_W1 = 48


def _make_seg_sum(width):
  mesh = plsc.VectorSubcoreMesh(core_axis_name="c", subcore_axis_name="s",
                                num_cores=_NC, num_subcores=_NS)

  @functools.partial(
      pl.kernel,
      mesh=mesh,
      compiler_params=pltpu.CompilerParams(use_tc_tiling_on_sc=False),
      out_type=jax.ShapeDtypeStruct((_NC, _NACC, width), jnp.float32),
      scratch_types=[
          pltpu.VMEM((_NBLK, _K), jnp.int32),
          pltpu.VMEM((_NBLK, _K), jnp.int32),
          pltpu.VMEM((_K, width), jnp.float32),
          pltpu.VMEM_SHARED((_NACC, width), jnp.float32),
          pltpu.SemaphoreType.DMA,
      ],
  )
  def seg(table_hbm, src_hbm, dst_hbm, zeros_hbm, out_hbm,
          src_v, dst_v, rows_v, acc_sh, sem):
    cid = lax.axis_index("c")
    sid = lax.axis_index("s")

    pltpu.sync_copy(zeros_hbm, acc_sh.at[pl.ds(sid * _RPT, _RPT)])
    pltpu.sync_copy(src_hbm.at[cid, sid], src_v)
    pltpu.sync_copy(dst_hbm.at[cid, sid], dst_v)
    plsc.subcore_barrier()

    def body(j, carry):
      pltpu.async_copy(table_hbm.at[src_v.at[j]], rows_v, sem).wait()
      pltpu.sync_copy(rows_v, acc_sh.at[dst_v.at[j]], add=True)
      return carry

    lax.fori_loop(0, _NBLK, body, 0)

    plsc.subcore_barrier()
    pltpu.sync_copy(acc_sh.at[pl.ds(sid * _RPT, _RPT)],
                    out_hbm.at[cid, pl.ds(sid * _RPT, _RPT)])

  return seg


_seg_sum_cache = {}


def _seg_sum(width):
  if width not in _seg_sum_cache:
    _seg_sum_cache[width] = _make_seg_sum(width)
  return _seg_sum_cache[width]


_R = 1000


def _tableA_body(x_ref, wn0_ref, o_ref):
  xw = jnp.dot(x_ref[...], wn0_ref[...], preferred_element_type=jnp.float32)
  col = lax.broadcasted_iota(jnp.int32, (_R, _W0 - _D), 1)
  ones = jnp.where(col == 0, 1.0, 0.0).astype(jnp.float32)
  o_ref[...] = jnp.concatenate([xw, ones], axis=1)


def _layerB_body(x_ref, p0_ref, ws0_ref, b0_ref, ws1_ref, wn1_ref, b1_ref,
                 t1_ref, hse_ref):
  agg = p0_ref[0] + p0_ref[1]
  deg = agg[:, _D:_D + 1]
  inv = 1.0 / jnp.maximum(deg, 1.0)
  hn = agg[:, :_D] * inv
  h = x_ref[...] @ ws0_ref[...] + hn + b0_ref[...][None, :]
  h = jnp.maximum(h, 0.0)
  t1 = jnp.dot(h, wn1_ref[...], preferred_element_type=jnp.float32)
  hs = jnp.dot(h, ws1_ref[...], preferred_element_type=jnp.float32)
  hs = hs + b1_ref[...][None, :]
  zpad = jnp.zeros((_R, _W1 - _C - 1), jnp.float32)
  t1_ref[...] = jnp.concatenate([t1, jnp.zeros((_R, _W1 - _C), jnp.float32)],
                                axis=1)
  hse_ref[...] = jnp.concatenate([hs, inv, zpad], axis=1)


def _layerC_body(p1_ref, hse_ref, o_ref):
  s = p1_ref[0] + p1_ref[1]
  inv = hse_ref[:, _C:_C + 1]
  o_ref[...] = hse_ref[:, :_C] + s[:, :_C] * inv


def kernel(features, edge_index, W_self0, W_neigh0, b0, W_self1, W_neigh1, b1):
  src = edge_index[0]
  dst = edge_index[1]
  pad = _EPAD - _E
  src_p = jnp.concatenate([src, jnp.zeros((pad,), jnp.int32)])
  dst_p = jnp.concatenate([dst, jnp.full((pad,), _N, jnp.int32)])
  src_r = src_p.reshape(_NC, _NS, _NBLK, _K)
  dst_r = dst_p.reshape(_NC, _NS, _NBLK, _K)
  zeros0 = jnp.zeros((_RPT, _W0), jnp.float32)
  zeros1 = jnp.zeros((_RPT, _W1), jnp.float32)

  nb = _N // _R

  t0 = pl.pallas_call(
      _tableA_body,
      grid=(nb,),
      in_specs=[
          pl.BlockSpec((_R, _D), lambda i: (i, 0)),
          pl.BlockSpec((_D, _H), lambda i: (0, 0)),
      ],
      out_specs=pl.BlockSpec((_R, _W0), lambda i: (i, 0)),
      out_shape=jax.ShapeDtypeStruct((_N, _W0), jnp.float32),
  )(features, W_neigh0)

  p0 = _seg_sum(_W0)(t0, src_r, dst_r, zeros0)

  t1, hse = pl.pallas_call(
      _layerB_body,
      grid=(nb,),
      in_specs=[
          pl.BlockSpec((_R, _D), lambda i: (i, 0)),
          pl.BlockSpec((_NC, _R, _W0), lambda i: (0, i, 0)),
          pl.BlockSpec((_D, _H), lambda i: (0, 0)),
          pl.BlockSpec((_H,), lambda i: (0,)),
          pl.BlockSpec((_H, _C), lambda i: (0, 0)),
          pl.BlockSpec((_H, _C), lambda i: (0, 0)),
          pl.BlockSpec((_C,), lambda i: (0,)),
      ],
      out_specs=[
          pl.BlockSpec((_R, _W1), lambda i: (i, 0)),
          pl.BlockSpec((_R, _W1), lambda i: (i, 0)),
      ],
      out_shape=[
          jax.ShapeDtypeStruct((_N, _W1), jnp.float32),
          jax.ShapeDtypeStruct((_N, _W1), jnp.float32),
      ],
  )(features, p0, W_self0, b0, W_self1, W_neigh1, b1)

  p1 = _seg_sum(_W1)(t1, src_r, dst_r, zeros1)

  out = pl.pallas_call(
      _layerC_body,
      grid=(nb,),
      in_specs=[
          pl.BlockSpec((_NC, _R, _W1), lambda i: (0, i, 0)),
          pl.BlockSpec((_R, _W1), lambda i: (i, 0)),
      ],
      out_specs=pl.BlockSpec((_R, _C), lambda i: (i, 0)),
      out_shape=jax.ShapeDtypeStruct((_N, _C), jnp.float32),
  )(p1, hse)

  return out
